# Initial kernel scaffold; baseline (speedup 1.0000x reference)
import jax
import jax.numpy as jnp
from jax.experimental import pallas as pl


def kernel(species_index, embedding_table):
    raise NotImplementedError("write your pallas kernel here")


# speedup vs baseline: 1.6804x; 1.6804x over previous
import functools

import jax
import jax.numpy as jnp
from jax import lax
from jax.experimental import pallas as pl
from jax.experimental.pallas import tpu as pltpu
from jax.experimental.pallas import tpu_sc as plsc

NUM_SPECIES = 100
EMBED_DIM = 16
N_NODES = 100000

NC = 2
NS = 16
NW = NC * NS

CHUNK = 128
GROUP = 5
N_CHUNKS = 25
B_PER_W = CHUNK * N_CHUNKS
B_PAD = B_PER_W * NW


@functools.cache
def _make_gather():
    mesh = plsc.VectorSubcoreMesh(
        core_axis_name="c", subcore_axis_name="s", num_cores=NC, num_subcores=NS
    )

    @functools.partial(
        pl.kernel,
        out_type=jax.ShapeDtypeStruct((NW, B_PER_W, EMBED_DIM), jnp.float32),
        mesh=mesh,
        scratch_types=[
            pltpu.VMEM((N_CHUNKS, CHUNK), jnp.int32),
            pltpu.VMEM((B_PER_W, EMBED_DIM), jnp.float32),
            pltpu.SemaphoreType.DMA,
        ],
        compiler_params=pltpu.CompilerParams(use_tc_tiling_on_sc=False),
    )
    def gather_kernel(table_hbm, idx_hbm, out_hbm, idx_v, rows_v, sem):
        wid = lax.axis_index("s") * NC + lax.axis_index("c")
        pltpu.sync_copy(idx_hbm.at[wid], idx_v)

        def group_body(g, carry):
            copies = []
            for b in range(GROUP):
                j = g * GROUP + b
                copies.append(
                    pltpu.async_copy(
                        table_hbm.at[idx_v.at[j]],
                        rows_v.at[pl.ds(j * CHUNK, CHUNK)],
                        sem,
                    )
                )
            for c in copies:
                c.wait()
            return carry

        lax.fori_loop(0, N_CHUNKS // GROUP, group_body, 0)
        pltpu.sync_copy(rows_v, out_hbm.at[wid])

    return gather_kernel


@jax.jit
def kernel(species_index, embedding_table):
    idx = species_index.astype(jnp.int32)
    idx = jnp.concatenate([idx, jnp.zeros((B_PAD - N_NODES,), jnp.int32)])
    idx = idx.reshape(NW, N_CHUNKS, CHUNK)
    out = _make_gather()(embedding_table, idx)
    return out.reshape(B_PAD, EMBED_DIM)[:N_NODES]


# speedup vs baseline: 1.6924x; 1.0071x over previous
import functools

import jax
import jax.numpy as jnp
from jax import lax
from jax.experimental import pallas as pl
from jax.experimental.pallas import tpu as pltpu
from jax.experimental.pallas import tpu_sc as plsc

NUM_SPECIES = 100
EMBED_DIM = 16
N_NODES = 100000

NC = 2
NS = 16
NW = NC * NS

CHUNK = 128
GROUP = 5
N_CHUNKS = 25
B_PER_W = CHUNK * N_CHUNKS
B_PAD = B_PER_W * NW


@functools.cache
def _make_gather():
    mesh = plsc.VectorSubcoreMesh(
        core_axis_name="c", subcore_axis_name="s", num_cores=NC, num_subcores=NS
    )

    @functools.partial(
        pl.kernel,
        out_type=jax.ShapeDtypeStruct((NW, B_PER_W, EMBED_DIM), jnp.float32),
        mesh=mesh,
        scratch_types=[
            pltpu.VMEM((B_PER_W,), jnp.int32),
            pltpu.VMEM((B_PER_W, EMBED_DIM), jnp.float32),
            pltpu.SemaphoreType.DMA,
        ],
        compiler_params=pltpu.CompilerParams(use_tc_tiling_on_sc=False),
    )
    def gather_kernel(table_hbm, idx_hbm, out_hbm, idx_v, rows_v, sem):
        wid = lax.axis_index("s") * NC + lax.axis_index("c")
        pltpu.sync_copy(idx_hbm.at[wid], idx_v)
        pltpu.async_copy(table_hbm.at[idx_v], rows_v, sem).wait()
        pltpu.sync_copy(rows_v, out_hbm.at[wid])

    return gather_kernel


@jax.jit
def kernel(species_index, embedding_table):
    idx = species_index.astype(jnp.int32)
    idx = jnp.concatenate([idx, jnp.zeros((B_PAD - N_NODES,), jnp.int32)])
    idx = idx.reshape(NW, B_PER_W)
    out = _make_gather()(embedding_table, idx)
    return out.reshape(B_PAD, EMBED_DIM)[:N_NODES]


# speedup vs baseline: 3.1436x; 1.8575x over previous
import functools

import jax
import jax.numpy as jnp
from jax import lax
from jax.experimental import pallas as pl
from jax.experimental.pallas import tpu as pltpu
from jax.experimental.pallas import tpu_sc as plsc

NUM_SPECIES = 100
EMBED_DIM = 16
N_NODES = 100000

NC = 2
NS = 16
NW = NC * NS

B_MAIN = 3136
B_TAIL = N_NODES - 31 * B_MAIN


@functools.cache
def _make_gather():
    mesh = plsc.VectorSubcoreMesh(
        core_axis_name="c", subcore_axis_name="s", num_cores=NC, num_subcores=NS
    )

    @functools.partial(
        pl.kernel,
        out_type=jax.ShapeDtypeStruct((N_NODES, EMBED_DIM), jnp.float32),
        mesh=mesh,
        scratch_types=[
            pltpu.VMEM((NUM_SPECIES, EMBED_DIM), jnp.float32),
            pltpu.VMEM((B_MAIN,), jnp.int32),
            pltpu.VMEM((B_MAIN, EMBED_DIM), jnp.float32),
        ],
        compiler_params=pltpu.CompilerParams(use_tc_tiling_on_sc=False, needs_layout_passes=False),
    )
    def gather_kernel(table_hbm, idx_hbm, out_hbm, table_v, idx_v, rows_v):
        wid = lax.axis_index("s") * NC + lax.axis_index("c")
        base = wid * B_MAIN
        pltpu.sync_copy(table_hbm, table_v)

        lanes = lax.iota(jnp.int32, 16)

        def do_chunk(n_rows):
            pltpu.sync_copy(
                idx_hbm.at[pl.ds(base, n_rows)], idx_v.at[pl.ds(0, n_rows)]
            )

            def group_body(g, carry):
                row0 = g * 16
                idx16 = plsc.load_gather(idx_v, [row0 + lanes])
                for r in range(16):
                    sidx = lax.gather(
                        idx16,
                        jnp.full((16, 1), r, jnp.int32),
                        lax.GatherDimensionNumbers(
                            offset_dims=(),
                            collapsed_slice_dims=(0,),
                            start_index_map=(0,),
                        ),
                        (1,),
                        mode=lax.GatherScatterMode.PROMISE_IN_BOUNDS,
                    )
                    v = plsc.load_gather(table_v, [sidx, lanes])
                    plsc.store_scatter(
                        rows_v, [jnp.full((16,), row0 + r, jnp.int32), lanes], v
                    )
                return carry

            lax.fori_loop(0, n_rows // 16, group_body, 0)
            pltpu.sync_copy(
                rows_v.at[pl.ds(0, n_rows)], out_hbm.at[pl.ds(base, n_rows)]
            )

        @pl.when(wid < NW - 1)
        def _():
            do_chunk(B_MAIN)

        @pl.when(wid == NW - 1)
        def _():
            do_chunk(B_TAIL)

    return gather_kernel


@jax.jit
def kernel(species_index, embedding_table):
    return _make_gather()(embedding_table, species_index.astype(jnp.int32))


# speedup vs baseline: 3.2076x; 1.0204x over previous
import functools

import jax
import jax.numpy as jnp
from jax import lax
from jax.experimental import pallas as pl
from jax.experimental.pallas import tpu as pltpu
from jax.experimental.pallas import tpu_sc as plsc

NUM_SPECIES = 100
EMBED_DIM = 16
N_NODES = 100000

NC = 2
NS = 16
NW = NC * NS

B_MAIN = 3136
B_TAIL = N_NODES - 31 * B_MAIN


@functools.cache
def _make_gather():
    mesh = plsc.VectorSubcoreMesh(
        core_axis_name="c", subcore_axis_name="s", num_cores=NC, num_subcores=NS
    )

    @functools.partial(
        pl.kernel,
        out_type=jax.ShapeDtypeStruct((N_NODES, EMBED_DIM), jnp.float32),
        mesh=mesh,
        scratch_types=[
            pltpu.VMEM((NUM_SPECIES, EMBED_DIM), jnp.float32),
            pltpu.VMEM((B_MAIN,), jnp.int32),
            pltpu.VMEM((B_MAIN, EMBED_DIM), jnp.float32),
            pltpu.SemaphoreType.DMA,
        ],
        compiler_params=pltpu.CompilerParams(use_tc_tiling_on_sc=False, needs_layout_passes=False),
    )
    def gather_kernel(table_hbm, idx_hbm, out_hbm, table_v, idx_v, rows_v, sem):
        wid = lax.axis_index("s") * NC + lax.axis_index("c")
        base = wid * B_MAIN
        pltpu.sync_copy(table_hbm, table_v)

        lanes = lax.iota(jnp.int32, 16)

        def do_rows(row0, count):
            def group_body(g, carry):
                for half in range(2):
                    gbase = row0 + g * 32 + half * 16
                    idx16 = plsc.load_gather(idx_v, [gbase + lanes])
                    for r in range(16):
                        sidx = lax.gather(
                            idx16,
                            jnp.full((16, 1), r, jnp.int32),
                            lax.GatherDimensionNumbers(
                                offset_dims=(),
                                collapsed_slice_dims=(0,),
                                start_index_map=(0,),
                            ),
                            (1,),
                            mode=lax.GatherScatterMode.PROMISE_IN_BOUNDS,
                        )
                        v = plsc.load_gather(table_v, [sidx, lanes])
                        rows_v[gbase + r, :] = v
                return carry

            lax.fori_loop(0, count // 32, group_body, 0)

        def do_chunk(n_rows):
            pltpu.sync_copy(
                idx_hbm.at[pl.ds(base, n_rows)], idx_v.at[pl.ds(0, n_rows)]
            )
            half = (n_rows // 64) * 32
            rest = n_rows - half
            do_rows(0, half)
            c0 = pltpu.async_copy(
                rows_v.at[pl.ds(0, half)], out_hbm.at[pl.ds(base, half)], sem
            )
            do_rows(half, rest)
            c1 = pltpu.async_copy(
                rows_v.at[pl.ds(half, rest)],
                out_hbm.at[pl.ds(base + half, rest)],
                sem,
            )
            c0.wait()
            c1.wait()

        @pl.when(wid < NW - 1)
        def _():
            do_chunk(B_MAIN)

        @pl.when(wid == NW - 1)
        def _():
            do_chunk(B_TAIL)

    return gather_kernel


@jax.jit
def kernel(species_index, embedding_table):
    return _make_gather()(embedding_table, species_index.astype(jnp.int32))


# speedup vs baseline: 7.0108x; 2.1857x over previous
import functools

import jax
import jax.numpy as jnp
from jax import lax
from jax.experimental import pallas as pl
from jax.experimental.pallas import tpu as pltpu
from jax.experimental.pallas import tpu_sc as plsc

NUM_SPECIES = 100
EMBED_DIM = 16
N_NODES = 100000

NC = 2
NS = 16
NW = NC * NS

B_MAIN = 3200
B_TAIL = N_NODES - 31 * B_MAIN


@functools.cache
def _make_gather():
    mesh = plsc.VectorSubcoreMesh(
        core_axis_name="c", subcore_axis_name="s", num_cores=NC, num_subcores=NS
    )

    @functools.partial(
        pl.kernel,
        out_type=jax.ShapeDtypeStruct((EMBED_DIM, N_NODES), jnp.float32),
        mesh=mesh,
        scratch_types=[
            pltpu.VMEM((NUM_SPECIES * EMBED_DIM,), jnp.float32),
            pltpu.VMEM((B_MAIN,), jnp.int32),
            pltpu.VMEM((EMBED_DIM, B_MAIN), jnp.float32),
        ],
        compiler_params=pltpu.CompilerParams(
            use_tc_tiling_on_sc=True, needs_layout_passes=False
        ),
    )
    def gather_kernel(table_hbm, idx_hbm, out_hbm, table_v, idx_v, col_v):
        wid = lax.axis_index("s") * NC + lax.axis_index("c")
        base = wid * B_MAIN
        pltpu.sync_copy(table_hbm, table_v)

        lanes = lax.iota(jnp.int32, 16)

        def do_chunk(n_rows):
            pltpu.sync_copy(
                idx_hbm.at[pl.ds(base, n_rows)], idx_v.at[pl.ds(0, n_rows)]
            )

            def group_body(g, carry):
                g16 = g * 16
                idx16 = plsc.load_gather(idx_v, [g16 + lanes])
                for c in range(EMBED_DIM):
                    v = plsc.load_gather(table_v, [idx16 + (c * NUM_SPECIES)])
                    col_v[c, pl.ds(g16, 16)] = v
                return carry

            lax.fori_loop(0, n_rows // 16, group_body, 0)
            n_dma = ((n_rows + 127) // 128) * 128
            pltpu.sync_copy(
                col_v.at[:, pl.ds(0, n_dma)],
                out_hbm.at[:, pl.ds(base, n_dma)],
            )

        @pl.when(wid < NW - 1)
        def _():
            do_chunk(B_MAIN)

        @pl.when(wid == NW - 1)
        def _():
            do_chunk(B_TAIL)

    return gather_kernel


@jax.jit
def kernel(species_index, embedding_table):
    table_t = embedding_table.T.reshape(NUM_SPECIES * EMBED_DIM)
    out_t = _make_gather()(table_t, species_index.astype(jnp.int32))
    return out_t.T


# speedup vs baseline: 9.4659x; 1.3502x over previous
import functools

import jax
import jax.numpy as jnp
from jax import lax
from jax.experimental import pallas as pl
from jax.experimental.pallas import tpu as pltpu
from jax.experimental.pallas import tpu_sc as plsc

NUM_SPECIES = 100
EMBED_DIM = 16
N_NODES = 100000

NC = 2
NS = 16
NW = NC * NS

B_MAIN = 3200
B_TAIL = N_NODES - 31 * B_MAIN


@functools.cache
def _make_gather():
    mesh = plsc.VectorSubcoreMesh(
        core_axis_name="c", subcore_axis_name="s", num_cores=NC, num_subcores=NS
    )

    @functools.partial(
        pl.kernel,
        out_type=jax.ShapeDtypeStruct((EMBED_DIM, N_NODES), jnp.float32),
        mesh=mesh,
        scratch_types=[
            pltpu.VMEM((NUM_SPECIES * EMBED_DIM,), jnp.float32),
            pltpu.VMEM((B_MAIN,), jnp.int32),
            pltpu.VMEM((EMBED_DIM, B_MAIN), jnp.float32),
        ],
        compiler_params=pltpu.CompilerParams(
            use_tc_tiling_on_sc=True, needs_layout_passes=False
        ),
    )
    def gather_kernel(table_hbm, idx_hbm, out_hbm, table_v, idx_v, col_v):
        wid = lax.axis_index("s") * NC + lax.axis_index("c")
        base = wid * B_MAIN
        pltpu.sync_copy(table_hbm, table_v)

        lanes = lax.iota(jnp.int32, 16)

        def do_chunk(n_rows):
            pltpu.sync_copy(
                idx_hbm.at[pl.ds(base, n_rows)], idx_v.at[pl.ds(0, n_rows)]
            )

            def group_body(g, carry):
                for sub in range(2):
                    g16 = (g * 2 + sub) * 16
                    idx16 = plsc.load_gather(idx_v, [g16 + lanes])
                    vs = [
                        plsc.load_gather(table_v, [idx16 + (c * NUM_SPECIES)])
                        for c in range(EMBED_DIM)
                    ]
                    for c in range(EMBED_DIM):
                        col_v[c, pl.ds(g16, 16)] = vs[c]
                return carry

            lax.fori_loop(0, n_rows // 32, group_body, 0)
            n_dma = ((n_rows + 127) // 128) * 128
            pltpu.sync_copy(
                col_v.at[:, pl.ds(0, n_dma)],
                out_hbm.at[:, pl.ds(base, n_dma)],
            )

        @pl.when(wid < NW - 1)
        def _():
            do_chunk(B_MAIN)

        @pl.when(wid == NW - 1)
        def _():
            do_chunk(B_TAIL)

    return gather_kernel


@jax.jit
def kernel(species_index, embedding_table):
    table_t = embedding_table.T.reshape(NUM_SPECIES * EMBED_DIM)
    out_t = _make_gather()(table_t, species_index.astype(jnp.int32))
    return out_t.T


# speedup vs baseline: 9.6913x; 1.0238x over previous
import functools

import jax
import jax.numpy as jnp
from jax import lax
from jax.experimental import pallas as pl
from jax.experimental.pallas import tpu as pltpu
from jax.experimental.pallas import tpu_sc as plsc

NUM_SPECIES = 100
EMBED_DIM = 16
N_NODES = 100000

NC = 2
NS = 16
NW = NC * NS

B_MAIN = 3200
B_TAIL = N_NODES - 31 * B_MAIN


@functools.cache
def _make_gather():
    mesh = plsc.VectorSubcoreMesh(
        core_axis_name="c", subcore_axis_name="s", num_cores=NC, num_subcores=NS
    )

    @functools.partial(
        pl.kernel,
        out_type=jax.ShapeDtypeStruct((EMBED_DIM, N_NODES), jnp.float32),
        mesh=mesh,
        scratch_types=[
            pltpu.VMEM((NUM_SPECIES * EMBED_DIM,), jnp.float32),
            pltpu.VMEM((B_MAIN,), jnp.int32),
            pltpu.VMEM((EMBED_DIM, B_MAIN), jnp.float32),
            pltpu.SemaphoreType.DMA,
        ],
        compiler_params=pltpu.CompilerParams(
            use_tc_tiling_on_sc=True, needs_layout_passes=False
        ),
    )
    def gather_kernel(table_hbm, idx_hbm, out_hbm, table_v, idx_v, col_v, sem):
        wid = lax.axis_index("s") * NC + lax.axis_index("c")
        base = wid * B_MAIN
        pltpu.sync_copy(table_hbm, table_v)

        def load_group(g16):
            idx16 = idx_v[pl.ds(g16, 16)]
            return tuple(
                plsc.load_gather(table_v, [idx16 + (c * NUM_SPECIES)])
                for c in range(EMBED_DIM)
            )

        def store_group(g16, vs):
            for c in range(EMBED_DIM):
                col_v[c, pl.ds(g16, 16)] = vs[c]

        def do_rows(row0, count):
            def group_body(g, carry):
                prev_g16, prev = carry
                g16 = row0 + g * 16
                new = load_group(g16)
                store_group(prev_g16, prev)
                return (g16, new)

            carry = (row0, load_group(row0))
            carry = lax.fori_loop(1, count // 16, group_body, carry)
            store_group(*carry)

        def do_chunk(parts):
            n_rows = sum(parts)
            pltpu.sync_copy(
                idx_hbm.at[pl.ds(base, n_rows)], idx_v.at[pl.ds(0, n_rows)]
            )
            copies = []
            row0 = 0
            for part in parts:
                do_rows(row0, part)
                n_dma = ((part + 127) // 128) * 128
                copies.append(
                    pltpu.async_copy(
                        col_v.at[:, pl.ds(row0, n_dma)],
                        out_hbm.at[:, pl.ds(base + row0, n_dma)],
                        sem,
                    )
                )
                row0 += part
            for cp in copies:
                cp.wait()

        @pl.when(wid < NW - 1)
        def _():
            do_chunk((1664, 1536))

        @pl.when(wid == NW - 1)
        def _():
            do_chunk((B_TAIL,))

    return gather_kernel


@jax.jit
def kernel(species_index, embedding_table):
    table_t = embedding_table.T.reshape(NUM_SPECIES * EMBED_DIM)
    out_t = _make_gather()(table_t, species_index.astype(jnp.int32))
    return out_t.T


# speedup vs baseline: 9.7127x; 1.0022x over previous
import functools

import jax
import jax.numpy as jnp
from jax import lax
from jax.experimental import pallas as pl
from jax.experimental.pallas import tpu as pltpu
from jax.experimental.pallas import tpu_sc as plsc

NUM_SPECIES = 100
EMBED_DIM = 16
N_NODES = 100000

NC = 2
NS = 16
NW = NC * NS

B_MAIN = 3200
B_TAIL = N_NODES - 31 * B_MAIN


@functools.cache
def _make_gather():
    mesh = plsc.VectorSubcoreMesh(
        core_axis_name="c", subcore_axis_name="s", num_cores=NC, num_subcores=NS
    )

    @functools.partial(
        pl.kernel,
        out_type=jax.ShapeDtypeStruct((EMBED_DIM, N_NODES), jnp.float32),
        mesh=mesh,
        scratch_types=[
            pltpu.VMEM((NUM_SPECIES * EMBED_DIM,), jnp.float32),
            pltpu.VMEM((B_MAIN,), jnp.int32),
            pltpu.VMEM((EMBED_DIM, B_MAIN), jnp.float32),
            pltpu.SemaphoreType.DMA,
        ],
        compiler_params=pltpu.CompilerParams(
            use_tc_tiling_on_sc=True, needs_layout_passes=False
        ),
    )
    def gather_kernel(table_hbm, idx_hbm, out_hbm, table_v, idx_v, col_v, sem):
        wid = lax.axis_index("s") * NC + lax.axis_index("c")
        base = wid * B_MAIN
        pltpu.sync_copy(table_hbm, table_v)

        def load_group(g16):
            idx16 = idx_v[pl.ds(g16, 16)]
            return tuple(
                plsc.load_gather(table_v, [idx16 + (c * NUM_SPECIES)])
                for c in range(EMBED_DIM)
            )

        def store_group(g16, vs):
            for c in range(EMBED_DIM):
                col_v[c, pl.ds(g16, 16)] = vs[c]

        def do_rows(row0, count):
            def group_body(g, carry):
                prev_g16, prev = carry
                g16 = row0 + g * 16
                new = load_group(g16)
                store_group(prev_g16, prev)
                return (g16, new)

            carry = (row0, load_group(row0))
            carry = lax.fori_loop(1, count // 16, group_body, carry)
            store_group(*carry)

        def do_chunk(parts):
            n_rows = sum(parts)
            pltpu.sync_copy(
                idx_hbm.at[pl.ds(base, n_rows)], idx_v.at[pl.ds(0, n_rows)]
            )
            copies = []
            row0 = 0
            for part in parts:
                do_rows(row0, part)
                n_dma = ((part + 127) // 128) * 128
                copies.append(
                    pltpu.async_copy(
                        col_v.at[:, pl.ds(row0, n_dma)],
                        out_hbm.at[:, pl.ds(base + row0, n_dma)],
                        sem,
                    )
                )
                row0 += part
            for cp in copies:
                cp.wait()

        @pl.when(wid < NW - 1)
        def _():
            do_chunk((896, 768, 768, 768))

        @pl.when(wid == NW - 1)
        def _():
            do_chunk((B_TAIL,))

    return gather_kernel


@jax.jit
def kernel(species_index, embedding_table):
    table_t = embedding_table.T.reshape(NUM_SPECIES * EMBED_DIM)
    out_t = _make_gather()(table_t, species_index.astype(jnp.int32))
    return out_t.T
